# baseline (speedup 1.0000x reference)
import functools

import jax
import jax.numpy as jnp
from jax import lax
from jax.experimental import pallas as pl
from jax.experimental.pallas import tpu as pltpu
from jax.experimental.pallas import tpu_sc as plsc

_LANES = 16
_NTILES = 32
_BLK_E = 2000
_VPB = _BLK_E // _LANES
_UNROLL = 5


def _make_sc_partials(n_elems: int):
    assert n_elems % _BLK_E == 0
    n_blocks = n_elems // _BLK_E
    blocks_per_tile = -(-n_blocks // _NTILES)

    mesh = plsc.VectorSubcoreMesh(core_axis_name="c", subcore_axis_name="s")

    @functools.partial(
        pl.kernel,
        mesh=mesh,
        out_type=jax.ShapeDtypeStruct((_NTILES, 3 * _LANES), jnp.float32),
        compiler_params=pltpu.CompilerParams(needs_layout_passes=False),
        scratch_types=[
            pltpu.VMEM((2 * _BLK_E,), jnp.float32),
            pltpu.VMEM((2 * _BLK_E,), jnp.float32),
            pltpu.VMEM((_BLK_E,), jnp.int32),
            pltpu.VMEM((_BLK_E,), jnp.int32),
            pltpu.VMEM((3 * _LANES,), jnp.float32),
            pltpu.SemaphoreType.DMA,
            pltpu.SemaphoreType.DMA,
            pltpu.SemaphoreType.DMA,
            pltpu.SemaphoreType.DMA,
        ],
    )
    def sc_partials(pred_hbm, lab_hbm, out_hbm, pbuf0, pbuf1, lbuf0, lbuf1,
                    stag, sp0, sp1, sl0, sl1):
        wid = lax.axis_index("c") * 16 + lax.axis_index("s")
        pbufs, lbufs = (pbuf0, pbuf1), (lbuf0, lbuf1)
        psems, lsems = (sp0, sp1), (sl0, sl1)

        iota = lax.iota(jnp.int32, _LANES)
        iota2 = iota * 2
        zero = jnp.zeros((_LANES,), jnp.float32)

        def start_dma(i):
            g = jnp.minimum(wid + _NTILES * i, n_blocks - 1)
            hp = pltpu.async_copy(
                pred_hbm.at[pl.ds(g * (2 * _BLK_E), 2 * _BLK_E)],
                pbufs[i % 2], psems[i % 2])
            hl = pltpu.async_copy(
                lab_hbm.at[pl.ds(g * _BLK_E, _BLK_E)],
                lbufs[i % 2], lsems[i % 2])
            return hp, hl

        def block_partials(pbuf, lbuf):
            def body(k0, carry):
                p_all, p_pos, p_cnt = carry
                for u in range(_UNROLL):
                    k = k0 * _UNROLL + u
                    lv = lbuf[pl.ds(k * _LANES, _LANES)]
                    idx = iota2 + lv + k * (2 * _LANES)
                    c = plsc.load_gather(pbuf, [idx])
                    lvf = lv.astype(jnp.float32)
                    p_all = p_all + c
                    p_pos = p_pos + lvf * c
                    p_cnt = p_cnt + lvf
                return p_all, p_pos, p_cnt
            return lax.fori_loop(0, _VPB // _UNROLL, body, (zero, zero, zero))

        acc_all, acc_pos, acc_cnt = zero, zero, zero
        handles = [start_dma(0), start_dma(1)]
        for i in range(blocks_per_tile):
            hp, hl = handles[i % 2]
            hp.wait()
            hl.wait()
            p_all, p_pos, p_cnt = block_partials(pbufs[i % 2], lbufs[i % 2])
            gate = jnp.where(wid + _NTILES * i < n_blocks,
                             jnp.float32(1.0), jnp.float32(0.0))
            acc_all = acc_all + gate * p_all
            acc_pos = acc_pos + gate * p_pos
            acc_cnt = acc_cnt + gate * p_cnt
            if i + 2 < blocks_per_tile:
                handles[i % 2] = start_dma(i + 2)

        stag[pl.ds(0, _LANES)] = acc_all
        stag[pl.ds(_LANES, _LANES)] = acc_pos
        stag[pl.ds(2 * _LANES, _LANES)] = acc_cnt
        pltpu.sync_copy(stag, out_hbm.at[wid])

    return sc_partials


def kernel(pred, label):
    predf = pred.reshape(-1)
    lab = label.reshape(-1).astype(jnp.int32)
    n = lab.shape[0]

    parts = _make_sc_partials(n)(predf, lab)
    s = jnp.sum(parts.reshape(_NTILES, 3, _LANES), axis=(0, 2))
    sum_all, sum_pos, cnt_pos = s[0], s[1], s[2]
    sum_neg = sum_all - sum_pos
    cnt_neg = jnp.float32(n) - cnt_pos
    loss_pos = jnp.where(cnt_pos > 0, -sum_pos / jnp.maximum(cnt_pos, 1.0), 0.0)
    loss_neg = jnp.where(cnt_neg > 0, -sum_neg / jnp.maximum(cnt_neg, 1.0), 0.0)
    return loss_pos * 0.5 + loss_neg * 0.5


# speedup vs baseline: 16.7880x; 16.7880x over previous
import functools

import jax
import jax.numpy as jnp
from jax import lax
from jax.experimental import pallas as pl
from jax.experimental.pallas import tpu as pltpu
from jax.experimental.pallas import tpu_sc as plsc

_LANES = 16
_NTILES = 32
_BLK_E = 2000
_VPB = _BLK_E // _LANES
_UNROLL = 5


def _make_sc_partials(n_elems: int):
    assert n_elems % _BLK_E == 0
    n_blocks = n_elems // _BLK_E
    blocks_per_tile = -(-n_blocks // _NTILES)

    mesh = plsc.VectorSubcoreMesh(core_axis_name="c", subcore_axis_name="s")

    @functools.partial(
        pl.kernel,
        mesh=mesh,
        out_type=jax.ShapeDtypeStruct((_NTILES, 4 * _LANES), jnp.float32),
        compiler_params=pltpu.CompilerParams(
            needs_layout_passes=False, use_tc_tiling_on_sc=False),
        scratch_types=[
            pltpu.VMEM((_BLK_E,), jnp.float32),
            pltpu.VMEM((_BLK_E,), jnp.float32),
            pltpu.VMEM((_BLK_E,), jnp.float32),
            pltpu.VMEM((_BLK_E,), jnp.float32),
            pltpu.VMEM((_BLK_E,), jnp.int32),
            pltpu.VMEM((_BLK_E,), jnp.int32),
            pltpu.VMEM((4 * _LANES,), jnp.float32),
            pltpu.SemaphoreType.DMA,
            pltpu.SemaphoreType.DMA,
            pltpu.SemaphoreType.DMA,
            pltpu.SemaphoreType.DMA,
            pltpu.SemaphoreType.DMA,
            pltpu.SemaphoreType.DMA,
        ],
    )
    def sc_partials(p0_hbm, p1_hbm, lab_hbm, out_hbm,
                    a0, a1, b0, b1, l0, l1, stag,
                    sa0, sa1, sb0, sb1, sl0, sl1):
        wid = lax.axis_index("c") * 16 + lax.axis_index("s")
        abufs, bbufs, lbufs = (a0, a1), (b0, b1), (l0, l1)
        asems, bsems, lsems = (sa0, sa1), (sb0, sb1), (sl0, sl1)

        zero = jnp.zeros((_LANES,), jnp.float32)

        def start_dma(i):
            g = jnp.minimum(wid + _NTILES * i, n_blocks - 1)
            sl = pl.ds(g * _BLK_E, _BLK_E)
            ha = pltpu.async_copy(p0_hbm.at[sl], abufs[i % 2], asems[i % 2])
            hb = pltpu.async_copy(p1_hbm.at[sl], bbufs[i % 2], bsems[i % 2])
            hl = pltpu.async_copy(lab_hbm.at[sl], lbufs[i % 2], lsems[i % 2])
            return ha, hb, hl

        def block_partials(abuf, bbuf, lbuf):
            def body(k0, carry):
                s_p1l, s_p0, s_p0l, s_cnt = carry
                for u in range(_UNROLL):
                    k = (k0 * _UNROLL + u) * _LANES
                    lv = lbuf[pl.ds(k, _LANES)]
                    v0 = abuf[pl.ds(k, _LANES)]
                    v1 = bbuf[pl.ds(k, _LANES)]
                    lvf = lv.astype(jnp.float32)
                    s_p1l = s_p1l + lvf * v1
                    s_p0 = s_p0 + v0
                    s_p0l = s_p0l + lvf * v0
                    s_cnt = s_cnt + lvf
                return s_p1l, s_p0, s_p0l, s_cnt
            return lax.fori_loop(0, _VPB // _UNROLL, body,
                                 (zero, zero, zero, zero))

        accs = [zero, zero, zero, zero]
        handles = [start_dma(0), start_dma(1)]
        for i in range(blocks_per_tile):
            for h in handles[i % 2]:
                h.wait()
            parts = block_partials(abufs[i % 2], bbufs[i % 2], lbufs[i % 2])
            gate = jnp.where(wid + _NTILES * i < n_blocks,
                             jnp.float32(1.0), jnp.float32(0.0))
            accs = [a + gate * p for a, p in zip(accs, parts)]
            if i + 2 < blocks_per_tile:
                handles[i % 2] = start_dma(i + 2)

        for j, a in enumerate(accs):
            stag[pl.ds(j * _LANES, _LANES)] = a
        pltpu.sync_copy(stag, out_hbm.at[wid])

    return sc_partials


def kernel(pred, label):
    lab = label.reshape(-1).astype(jnp.int32)
    n = lab.shape[0]
    pred2 = pred.reshape(n, 2)
    p0 = pred2[:, 0]
    p1 = pred2[:, 1]

    parts = _make_sc_partials(n)(p0, p1, lab)
    s = jnp.sum(parts.reshape(_NTILES, 4, _LANES), axis=(0, 2))
    sum_pos, sum_p0, sum_p0l, cnt_pos = s[0], s[1], s[2], s[3]
    sum_neg = sum_p0 - sum_p0l
    cnt_neg = jnp.float32(n) - cnt_pos
    loss_pos = jnp.where(cnt_pos > 0, -sum_pos / jnp.maximum(cnt_pos, 1.0), 0.0)
    loss_neg = jnp.where(cnt_neg > 0, -sum_neg / jnp.maximum(cnt_neg, 1.0), 0.0)
    return loss_pos * 0.5 + loss_neg * 0.5


# speedup vs baseline: 29.5229x; 1.7586x over previous
import functools

import jax
import jax.numpy as jnp
from jax import lax
from jax.experimental import pallas as pl
from jax.experimental.pallas import tpu as pltpu
from jax.experimental.pallas import tpu_sc as plsc

_LANES = 16
_NTILES = 32
_BLK_E = 2000
_VPB = _BLK_E // _LANES
_UNROLL = 5


def _make_sc_partials(n_elems: int):
    assert n_elems % _BLK_E == 0
    n_blocks = n_elems // _BLK_E
    blocks_per_tile = -(-n_blocks // _NTILES)

    mesh = plsc.VectorSubcoreMesh(core_axis_name="c", subcore_axis_name="s")

    @functools.partial(
        pl.kernel,
        mesh=mesh,
        out_type=jax.ShapeDtypeStruct((_NTILES, 4 * _LANES), jnp.float32),
        compiler_params=pltpu.CompilerParams(
            needs_layout_passes=False, use_tc_tiling_on_sc=False),
        scratch_types=[
            pltpu.VMEM((_BLK_E,), jnp.float32),
            pltpu.VMEM((_BLK_E,), jnp.float32),
            pltpu.VMEM((_BLK_E,), jnp.float32),
            pltpu.VMEM((_BLK_E,), jnp.float32),
            pltpu.VMEM((_BLK_E,), jnp.int32),
            pltpu.VMEM((_BLK_E,), jnp.int32),
            pltpu.VMEM((4 * _LANES,), jnp.float32),
            pltpu.SemaphoreType.DMA,
            pltpu.SemaphoreType.DMA,
            pltpu.SemaphoreType.DMA,
            pltpu.SemaphoreType.DMA,
            pltpu.SemaphoreType.DMA,
            pltpu.SemaphoreType.DMA,
        ],
    )
    def sc_partials(p01_hbm, lab_hbm, out_hbm,
                    a0, a1, b0, b1, l0, l1, stag,
                    sa0, sa1, sb0, sb1, sl0, sl1):
        wid = lax.axis_index("c") * 16 + lax.axis_index("s")
        abufs, bbufs, lbufs = (a0, a1), (b0, b1), (l0, l1)
        asems, bsems, lsems = (sa0, sa1), (sb0, sb1), (sl0, sl1)

        zero = jnp.zeros((_LANES,), jnp.float32)

        def start_dma(i):
            g = jnp.minimum(wid + _NTILES * i, n_blocks - 1)
            ha = pltpu.async_copy(
                p01_hbm.at[pl.ds(g * _BLK_E, _BLK_E)],
                abufs[i % 2], asems[i % 2])
            hb = pltpu.async_copy(
                p01_hbm.at[pl.ds(n_elems + g * _BLK_E, _BLK_E)],
                bbufs[i % 2], bsems[i % 2])
            hl = pltpu.async_copy(
                lab_hbm.at[pl.ds(g * _BLK_E, _BLK_E)],
                lbufs[i % 2], lsems[i % 2])
            return ha, hb, hl

        def block_partials(abuf, bbuf, lbuf):
            def body(k0, carry):
                s_p1l, s_p0, s_p0l, s_cnt = carry
                for u in range(_UNROLL):
                    k = (k0 * _UNROLL + u) * _LANES
                    lv = lbuf[pl.ds(k, _LANES)]
                    v0 = abuf[pl.ds(k, _LANES)]
                    v1 = bbuf[pl.ds(k, _LANES)]
                    lvf = lv.astype(jnp.float32)
                    s_p1l = s_p1l + lvf * v1
                    s_p0 = s_p0 + v0
                    s_p0l = s_p0l + lvf * v0
                    s_cnt = s_cnt + lvf
                return s_p1l, s_p0, s_p0l, s_cnt
            return lax.fori_loop(0, _VPB // _UNROLL, body,
                                 (zero, zero, zero, zero))

        accs = [zero, zero, zero, zero]
        handles = [start_dma(0), start_dma(1)]
        for i in range(blocks_per_tile):
            for h in handles[i % 2]:
                h.wait()
            parts = block_partials(abufs[i % 2], bbufs[i % 2], lbufs[i % 2])
            gate = jnp.where(wid + _NTILES * i < n_blocks,
                             jnp.float32(1.0), jnp.float32(0.0))
            accs = [a + gate * p for a, p in zip(accs, parts)]
            if i + 2 < blocks_per_tile:
                handles[i % 2] = start_dma(i + 2)

        for j, a in enumerate(accs):
            stag[pl.ds(j * _LANES, _LANES)] = a
        pltpu.sync_copy(stag, out_hbm.at[wid])

    return sc_partials


def kernel(pred, label):
    lab = label.reshape(-1).astype(jnp.int32)
    n = lab.shape[0]
    p01 = pred.reshape(n, 2).T.reshape(-1)

    parts = _make_sc_partials(n)(p01, lab)
    s = jnp.sum(parts.reshape(_NTILES, 4, _LANES), axis=(0, 2))
    sum_pos, sum_p0, sum_p0l, cnt_pos = s[0], s[1], s[2], s[3]
    sum_neg = sum_p0 - sum_p0l
    cnt_neg = jnp.float32(n) - cnt_pos
    loss_pos = jnp.where(cnt_pos > 0, -sum_pos / jnp.maximum(cnt_pos, 1.0), 0.0)
    loss_neg = jnp.where(cnt_neg > 0, -sum_neg / jnp.maximum(cnt_neg, 1.0), 0.0)
    return loss_pos * 0.5 + loss_neg * 0.5


# speedup vs baseline: 31.9817x; 1.0833x over previous
import functools

import jax
import jax.numpy as jnp
from jax import lax
from jax.experimental import pallas as pl
from jax.experimental.pallas import tpu as pltpu
from jax.experimental.pallas import tpu_sc as plsc

_LANES = 16
_NTILES = 32
_BLK_E = 8000
_VPB = _BLK_E // _LANES
_UNROLL = 5


def _make_sc_partials(n_elems: int):
    assert n_elems % _BLK_E == 0
    n_blocks = n_elems // _BLK_E
    blocks_per_tile = -(-n_blocks // _NTILES)

    mesh = plsc.VectorSubcoreMesh(core_axis_name="c", subcore_axis_name="s")

    @functools.partial(
        pl.kernel,
        mesh=mesh,
        out_type=jax.ShapeDtypeStruct((_NTILES, 4 * _LANES), jnp.float32),
        compiler_params=pltpu.CompilerParams(
            needs_layout_passes=False, use_tc_tiling_on_sc=False),
        scratch_types=[
            pltpu.VMEM((_BLK_E,), jnp.float32),
            pltpu.VMEM((_BLK_E,), jnp.float32),
            pltpu.VMEM((_BLK_E,), jnp.float32),
            pltpu.VMEM((_BLK_E,), jnp.float32),
            pltpu.VMEM((_BLK_E,), jnp.int32),
            pltpu.VMEM((_BLK_E,), jnp.int32),
            pltpu.VMEM((4 * _LANES,), jnp.float32),
            pltpu.SemaphoreType.DMA,
            pltpu.SemaphoreType.DMA,
            pltpu.SemaphoreType.DMA,
            pltpu.SemaphoreType.DMA,
            pltpu.SemaphoreType.DMA,
            pltpu.SemaphoreType.DMA,
        ],
    )
    def sc_partials(p01_hbm, lab_hbm, out_hbm,
                    a0, a1, b0, b1, l0, l1, stag,
                    sa0, sa1, sb0, sb1, sl0, sl1):
        wid = lax.axis_index("c") * 16 + lax.axis_index("s")
        abufs, bbufs, lbufs = (a0, a1), (b0, b1), (l0, l1)
        asems, bsems, lsems = (sa0, sa1), (sb0, sb1), (sl0, sl1)

        zero = jnp.zeros((_LANES,), jnp.float32)

        def start_dma(i):
            g = jnp.minimum(wid + _NTILES * i, n_blocks - 1)
            ha = pltpu.async_copy(
                p01_hbm.at[pl.ds(g * _BLK_E, _BLK_E)],
                abufs[i % 2], asems[i % 2])
            hb = pltpu.async_copy(
                p01_hbm.at[pl.ds(n_elems + g * _BLK_E, _BLK_E)],
                bbufs[i % 2], bsems[i % 2])
            hl = pltpu.async_copy(
                lab_hbm.at[pl.ds(g * _BLK_E, _BLK_E)],
                lbufs[i % 2], lsems[i % 2])
            return ha, hb, hl

        def block_partials(abuf, bbuf, lbuf):
            zero4 = (zero, zero, zero, zero)

            def update(accs, k):
                s_p1l, s_p0, s_p0l, s_cnt = accs
                off = k * _LANES
                lv = lbuf[pl.ds(off, _LANES)]
                v0 = abuf[pl.ds(off, _LANES)]
                v1 = bbuf[pl.ds(off, _LANES)]
                lvf = lv.astype(jnp.float32)
                return (s_p1l + lvf * v1, s_p0 + v0,
                        s_p0l + lvf * v0, s_cnt + lvf)

            @plsc.parallel_loop(0, _VPB, step=2, unroll=_UNROLL,
                                carry=(zero4, zero4))
            def body(k, carry):
                ca, cb = carry
                return update(ca, k), update(cb, k + 1)

            ca, cb = body
            return tuple(a + b for a, b in zip(ca, cb))

        accs = [zero, zero, zero, zero]
        handles = [start_dma(0), start_dma(1)]
        for i in range(blocks_per_tile):
            for h in handles[i % 2]:
                h.wait()
            parts = block_partials(abufs[i % 2], bbufs[i % 2], lbufs[i % 2])
            gate = jnp.where(wid + _NTILES * i < n_blocks,
                             jnp.float32(1.0), jnp.float32(0.0))
            accs = [a + gate * p for a, p in zip(accs, parts)]
            if i + 2 < blocks_per_tile:
                handles[i % 2] = start_dma(i + 2)

        for j, a in enumerate(accs):
            stag[pl.ds(j * _LANES, _LANES)] = a
        pltpu.sync_copy(stag, out_hbm.at[wid])

    return sc_partials


def kernel(pred, label):
    lab = label.reshape(-1).astype(jnp.int32)
    n = lab.shape[0]
    p01 = pred.reshape(n, 2).T.reshape(-1)

    parts = _make_sc_partials(n)(p01, lab)
    s = jnp.sum(parts.reshape(_NTILES, 4, _LANES), axis=(0, 2))
    sum_pos, sum_p0, sum_p0l, cnt_pos = s[0], s[1], s[2], s[3]
    sum_neg = sum_p0 - sum_p0l
    cnt_neg = jnp.float32(n) - cnt_pos
    loss_pos = jnp.where(cnt_pos > 0, -sum_pos / jnp.maximum(cnt_pos, 1.0), 0.0)
    loss_neg = jnp.where(cnt_neg > 0, -sum_neg / jnp.maximum(cnt_neg, 1.0), 0.0)
    return loss_pos * 0.5 + loss_neg * 0.5
